# baseline (speedup 1.0000x reference)
import functools

import jax
import jax.numpy as jnp
from jax import lax
from jax.experimental import pallas as pl
from jax.experimental.pallas import tpu as pltpu
from jax.experimental.pallas import tpu_sc as plsc

N = 10000
E = 320000
D = 128
LEAKY_SLOPE = 0.01

NC = 2
NS = 16
NW = NC * NS
EP = E // NW
CHUNK = 80
NCHUNK = EP // CHUNK
NROWS = 10240
TRASH = N
RPT = NROWS // NS

_mesh = plsc.VectorSubcoreMesh(core_axis_name="c", subcore_axis_name="s")


@functools.partial(
    pl.kernel,
    out_type=(
        jax.ShapeDtypeStruct((NC * NROWS, 1), jnp.float32),
        jax.ShapeDtypeStruct((NC * NROWS, 1), jnp.float32),
    ),
    mesh=_mesh,
    scratch_types=[
        pltpu.VMEM((EP,), jnp.int32),
        pltpu.VMEM((EP,), jnp.int32),
        pltpu.VMEM((CHUNK,), jnp.int32),
        pltpu.VMEM((CHUNK,), jnp.int32),
        pltpu.VMEM((CHUNK, 1), jnp.float32),
        pltpu.VMEM_SHARED((NROWS, 1), jnp.float32),
        pltpu.VMEM_SHARED((NROWS, 1), jnp.float32),
    ],
)
def _deg_kernel(src_hbm, dst_hbm, ones_hbm, zeros1_hbm, dout_hbm, din_hbm,
                srcv, dstv, sidx, didx, onesv, sh_out, sh_in):
    cid = lax.axis_index("c")
    sid = lax.axis_index("s")
    wid = sid * NC + cid
    base = wid * EP

    pltpu.sync_copy(src_hbm.at[pl.ds(base, EP)], srcv)
    pltpu.sync_copy(dst_hbm.at[pl.ds(base, EP)], dstv)
    pltpu.sync_copy(ones_hbm, onesv)

    pltpu.sync_copy(zeros1_hbm.at[pl.ds(sid * RPT, RPT)],
                    sh_out.at[pl.ds(sid * RPT, RPT)])
    pltpu.sync_copy(zeros1_hbm.at[pl.ds(sid * RPT, RPT)],
                    sh_in.at[pl.ds(sid * RPT, RPT)])
    plsc.subcore_barrier()

    def body(j, carry):
        off = j * CHUNK
        for g in range(CHUNK // 16):
            s16 = srcv[pl.ds(off + g * 16, 16)]
            d16 = dstv[pl.ds(off + g * 16, 16)]
            m = s16 != d16
            sidx[pl.ds(g * 16, 16)] = jnp.where(m, s16, TRASH)
            didx[pl.ds(g * 16, 16)] = jnp.where(m, d16, TRASH)
        pltpu.sync_copy(onesv, sh_out.at[sidx], add=True)
        pltpu.sync_copy(onesv, sh_in.at[didx], add=True)
        return carry

    lax.fori_loop(0, NCHUNK, body, 0)
    plsc.subcore_barrier()

    out_off = cid * NROWS + sid * RPT
    pltpu.sync_copy(sh_out.at[pl.ds(sid * RPT, RPT)],
                    dout_hbm.at[pl.ds(out_off, RPT)])
    pltpu.sync_copy(sh_in.at[pl.ds(sid * RPT, RPT)],
                    din_hbm.at[pl.ds(out_off, RPT)])


@functools.partial(
    pl.kernel,
    out_type=jax.ShapeDtypeStruct((NC * NROWS, D), jnp.float32),
    mesh=_mesh,
    scratch_types=[
        pltpu.VMEM((EP,), jnp.int32),
        pltpu.VMEM((EP,), jnp.int32),
        pltpu.VMEM((CHUNK,), jnp.int32),
        pltpu.VMEM((CHUNK,), jnp.int32),
        pltpu.VMEM((CHUNK, D), jnp.float32),
        pltpu.VMEM((CHUNK, D), jnp.float32),
        pltpu.VMEM_SHARED((NROWS, D), jnp.float32),
        pltpu.SemaphoreType.DMA,
        pltpu.SemaphoreType.DMA,
    ],
)
def _agg_kernel(h_hbm, src_hbm, dst_hbm, zeros_hbm, agg_hbm,
                srcv, dstv, didx0, didx1, rows0, rows1, sh_agg, sem0, sem1):
    cid = lax.axis_index("c")
    sid = lax.axis_index("s")
    wid = sid * NC + cid
    base = wid * EP

    pltpu.sync_copy(src_hbm.at[pl.ds(base, EP)], srcv)
    pltpu.sync_copy(dst_hbm.at[pl.ds(base, EP)], dstv)

    pltpu.sync_copy(zeros_hbm.at[pl.ds(sid * RPT, RPT)],
                    sh_agg.at[pl.ds(sid * RPT, RPT)])
    plsc.subcore_barrier()

    def gather(j, rows, sem):
        return pltpu.async_copy(
            h_hbm.at[srcv.at[pl.ds(j * CHUNK, CHUNK)]], rows, sem)

    def scatter(j, rows, didx):
        off = j * CHUNK
        for g in range(CHUNK // 16):
            s16 = srcv[pl.ds(off + g * 16, 16)]
            d16 = dstv[pl.ds(off + g * 16, 16)]
            m = s16 != d16
            didx[pl.ds(g * 16, 16)] = jnp.where(m, d16, TRASH)
        pltpu.sync_copy(rows, sh_agg.at[didx], add=True)

    gather(0, rows0, sem0)

    def body(t, carry):
        jb = t * 2
        gather(jb + 1, rows1, sem1)
        pltpu.make_async_copy(
            h_hbm.at[srcv.at[pl.ds(jb * CHUNK, CHUNK)]], rows0, sem0).wait()
        scatter(jb, rows0, didx0)
        g2 = gather(jb + 2, rows0, sem0)
        pltpu.make_async_copy(
            h_hbm.at[srcv.at[pl.ds((jb + 1) * CHUNK, CHUNK)]], rows1, sem1).wait()
        scatter(jb + 1, rows1, didx1)
        del g2
        return carry

    lax.fori_loop(0, (NCHUNK - 1) // 2, body, 0)
    pltpu.make_async_copy(
        h_hbm.at[srcv.at[pl.ds((NCHUNK - 1) * CHUNK, CHUNK)]], rows0, sem0).wait()
    scatter(NCHUNK - 1, rows0, didx0)
    plsc.subcore_barrier()

    out_off = cid * NROWS + sid * RPT
    pltpu.sync_copy(sh_agg.at[pl.ds(sid * RPT, RPT)],
                    agg_hbm.at[pl.ds(out_off, RPT)])


_BM = 1000
_BMF = 2000


def _mm_scale_body(x_ref, w_ref, dout_ref, h_ref):
    xw = jnp.dot(x_ref[...], w_ref[...], preferred_element_type=jnp.float32)
    deg = dout_ref[0, :, 0] + dout_ref[1, :, 0] + 1.0
    h_ref[...] = xw * lax.rsqrt(deg)[:, None]


def _mm_scale(x, W, dout):
    return pl.pallas_call(
        _mm_scale_body,
        grid=(N // _BM,),
        in_specs=[
            pl.BlockSpec((_BM, D), lambda i: (i, 0)),
            pl.BlockSpec((D, D), lambda i: (0, 0)),
            pl.BlockSpec((NC, _BM, 1), lambda i: (0, i, 0)),
        ],
        out_specs=pl.BlockSpec((_BM, D), lambda i: (i, 0)),
        out_shape=jax.ShapeDtypeStruct((N, D), jnp.float32),
    )(x, W, dout)


def _final_body(agg_ref, h_ref, din_ref, b_ref, o_ref):
    deg = din_ref[0, :, 0] + din_ref[1, :, 0] + 1.0
    s = agg_ref[0] + agg_ref[1] + h_ref[...]
    out = s * lax.rsqrt(deg)[:, None] + b_ref[0, :]
    o_ref[...] = jnp.where(out >= 0, out, LEAKY_SLOPE * out)


def _final(agg, h, din, b):
    return pl.pallas_call(
        _final_body,
        grid=(N // _BMF,),
        in_specs=[
            pl.BlockSpec((NC, _BMF, D), lambda i: (0, i, 0)),
            pl.BlockSpec((_BMF, D), lambda i: (i, 0)),
            pl.BlockSpec((NC, _BMF, 1), lambda i: (0, i, 0)),
            pl.BlockSpec((1, D), lambda i: (0, 0)),
        ],
        out_specs=pl.BlockSpec((_BMF, D), lambda i: (i, 0)),
        out_shape=jax.ShapeDtypeStruct((N, D), jnp.float32),
    )(agg, h, din, b)


def kernel(x, edge_index, W, b):
    src = edge_index[0]
    dst = edge_index[1]
    zeros = jnp.zeros((NROWS, D), jnp.float32)
    zeros1 = jnp.zeros((NROWS, 1), jnp.float32)
    ones = jnp.ones((CHUNK, 1), jnp.float32)

    dout, din = _deg_kernel(src, dst, ones, zeros1)
    dout = dout.reshape(NC, NROWS, 1)
    din = din.reshape(NC, NROWS, 1)

    h = _mm_scale(x, W, dout)
    agg = _agg_kernel(h, src, dst, zeros)
    agg = agg.reshape(NC, NROWS, D)
    out = _final(agg, h, din, b.reshape(1, D))
    return out


# speedup vs baseline: 1.0760x; 1.0760x over previous
import functools

import jax
import jax.numpy as jnp
from jax import lax
from jax.experimental import pallas as pl
from jax.experimental.pallas import tpu as pltpu
from jax.experimental.pallas import tpu_sc as plsc

N = 10000
E = 320000
D = 128
LEAKY_SLOPE = 0.01

NC = 2
NS = 16
NW = NC * NS
EP = E // NW
CHUNK = 80
NCHUNK = EP // CHUNK
NROWS = 10240
TRASH = N
RPT = NROWS // NS

_mesh = plsc.VectorSubcoreMesh(core_axis_name="c", subcore_axis_name="s")


@functools.partial(
    pl.kernel,
    out_type=(
        jax.ShapeDtypeStruct((NC * NROWS,), jnp.float32),
        jax.ShapeDtypeStruct((NC * NROWS,), jnp.float32),
    ),
    mesh=_mesh,
    scratch_types=[
        pltpu.VMEM((EP,), jnp.int32),
        pltpu.VMEM((EP,), jnp.int32),
        pltpu.VMEM((CHUNK,), jnp.int32),
        pltpu.VMEM((CHUNK,), jnp.int32),
        pltpu.VMEM((CHUNK,), jnp.float32),
        pltpu.VMEM((RPT,), jnp.float32),
        pltpu.VMEM_SHARED((NROWS,), jnp.float32),
        pltpu.VMEM_SHARED((NROWS,), jnp.float32),
    ],
)
def _deg_kernel(src_hbm, dst_hbm, dout_hbm, din_hbm,
                srcv, dstv, sidx, didx, onesv, zv, sh_out, sh_in):
    cid = lax.axis_index("c")
    sid = lax.axis_index("s")
    wid = sid * NC + cid
    base = wid * EP

    pltpu.sync_copy(src_hbm.at[pl.ds(base, EP)], srcv)
    pltpu.sync_copy(dst_hbm.at[pl.ds(base, EP)], dstv)

    for g in range(RPT // 16):
        zv[pl.ds(g * 16, 16)] = jnp.zeros((16,), jnp.float32)
    pltpu.sync_copy(zv, sh_out.at[pl.ds(sid * RPT, RPT)])
    pltpu.sync_copy(zv, sh_in.at[pl.ds(sid * RPT, RPT)])
    for g in range(CHUNK // 16):
        onesv[pl.ds(g * 16, 16)] = jnp.ones((16,), jnp.float32)
    plsc.subcore_barrier()

    def body(j, carry):
        off = j * CHUNK
        for g in range(CHUNK // 16):
            s16 = srcv[pl.ds(off + g * 16, 16)]
            d16 = dstv[pl.ds(off + g * 16, 16)]
            m = s16 != d16
            sidx[pl.ds(g * 16, 16)] = jnp.where(m, s16, TRASH)
            didx[pl.ds(g * 16, 16)] = jnp.where(m, d16, TRASH)
        pltpu.sync_copy(onesv, sh_out.at[sidx], add=True)
        pltpu.sync_copy(onesv, sh_in.at[didx], add=True)
        return carry

    lax.fori_loop(0, NCHUNK, body, 0)
    plsc.subcore_barrier()

    out_off = cid * NROWS + sid * RPT
    pltpu.sync_copy(sh_out.at[pl.ds(sid * RPT, RPT)],
                    dout_hbm.at[pl.ds(out_off, RPT)])
    pltpu.sync_copy(sh_in.at[pl.ds(sid * RPT, RPT)],
                    din_hbm.at[pl.ds(out_off, RPT)])


@functools.partial(
    pl.kernel,
    out_type=jax.ShapeDtypeStruct((NC * NROWS, D), jnp.float32),
    mesh=_mesh,
    scratch_types=[
        pltpu.VMEM((EP,), jnp.int32),
        pltpu.VMEM((EP,), jnp.int32),
        pltpu.VMEM((CHUNK,), jnp.int32),
        pltpu.VMEM((CHUNK,), jnp.int32),
        pltpu.VMEM((CHUNK, D), jnp.float32),
        pltpu.VMEM((CHUNK, D), jnp.float32),
        pltpu.VMEM_SHARED((NROWS, D), jnp.float32),
        pltpu.SemaphoreType.DMA,
        pltpu.SemaphoreType.DMA,
    ],
)
def _agg_kernel(h_hbm, src_hbm, dst_hbm, zeros_hbm, agg_hbm,
                srcv, dstv, didx0, didx1, rows0, rows1, sh_agg, sem0, sem1):
    cid = lax.axis_index("c")
    sid = lax.axis_index("s")
    wid = sid * NC + cid
    base = wid * EP

    pltpu.sync_copy(src_hbm.at[pl.ds(base, EP)], srcv)
    pltpu.sync_copy(dst_hbm.at[pl.ds(base, EP)], dstv)

    pltpu.sync_copy(zeros_hbm.at[pl.ds(sid * RPT, RPT)],
                    sh_agg.at[pl.ds(sid * RPT, RPT)])
    plsc.subcore_barrier()

    def gather(j, rows, sem):
        return pltpu.async_copy(
            h_hbm.at[srcv.at[pl.ds(j * CHUNK, CHUNK)]], rows, sem)

    def scatter(j, rows, didx):
        off = j * CHUNK
        for g in range(CHUNK // 16):
            s16 = srcv[pl.ds(off + g * 16, 16)]
            d16 = dstv[pl.ds(off + g * 16, 16)]
            m = s16 != d16
            didx[pl.ds(g * 16, 16)] = jnp.where(m, d16, TRASH)
        pltpu.sync_copy(rows, sh_agg.at[didx], add=True)

    gather(0, rows0, sem0)

    def body(t, carry):
        jb = t * 2
        gather(jb + 1, rows1, sem1)
        pltpu.make_async_copy(
            h_hbm.at[srcv.at[pl.ds(jb * CHUNK, CHUNK)]], rows0, sem0).wait()
        scatter(jb, rows0, didx0)
        g2 = gather(jb + 2, rows0, sem0)
        pltpu.make_async_copy(
            h_hbm.at[srcv.at[pl.ds((jb + 1) * CHUNK, CHUNK)]], rows1, sem1).wait()
        scatter(jb + 1, rows1, didx1)
        del g2
        return carry

    lax.fori_loop(0, (NCHUNK - 1) // 2, body, 0)
    pltpu.make_async_copy(
        h_hbm.at[srcv.at[pl.ds((NCHUNK - 1) * CHUNK, CHUNK)]], rows0, sem0).wait()
    scatter(NCHUNK - 1, rows0, didx0)
    plsc.subcore_barrier()

    out_off = cid * NROWS + sid * RPT
    pltpu.sync_copy(sh_agg.at[pl.ds(sid * RPT, RPT)],
                    agg_hbm.at[pl.ds(out_off, RPT)])


_BM = 1000
_BMF = 2000


def _mm_scale_body(x_ref, w_ref, dout_ref, h_ref):
    xw = jnp.dot(x_ref[...], w_ref[...], preferred_element_type=jnp.float32)
    deg = dout_ref[0, :, 0] + dout_ref[1, :, 0] + 1.0
    h_ref[...] = xw * lax.rsqrt(deg)[:, None]


def _mm_scale(x, W, dout):
    return pl.pallas_call(
        _mm_scale_body,
        grid=(N // _BM,),
        in_specs=[
            pl.BlockSpec((_BM, D), lambda i: (i, 0)),
            pl.BlockSpec((D, D), lambda i: (0, 0)),
            pl.BlockSpec((NC, _BM, 1), lambda i: (0, i, 0)),
        ],
        out_specs=pl.BlockSpec((_BM, D), lambda i: (i, 0)),
        out_shape=jax.ShapeDtypeStruct((N, D), jnp.float32),
    )(x, W, dout)


def _final_body(agg_ref, h_ref, din_ref, b_ref, o_ref):
    deg = din_ref[0, :, 0] + din_ref[1, :, 0] + 1.0
    s = agg_ref[0] + agg_ref[1] + h_ref[...]
    out = s * lax.rsqrt(deg)[:, None] + b_ref[0, :]
    o_ref[...] = jnp.where(out >= 0, out, LEAKY_SLOPE * out)


def _final(agg, h, din, b):
    return pl.pallas_call(
        _final_body,
        grid=(N // _BMF,),
        in_specs=[
            pl.BlockSpec((NC, _BMF, D), lambda i: (0, i, 0)),
            pl.BlockSpec((_BMF, D), lambda i: (i, 0)),
            pl.BlockSpec((NC, _BMF, 1), lambda i: (0, i, 0)),
            pl.BlockSpec((1, D), lambda i: (0, 0)),
        ],
        out_specs=pl.BlockSpec((_BMF, D), lambda i: (i, 0)),
        out_shape=jax.ShapeDtypeStruct((N, D), jnp.float32),
    )(agg, h, din, b)


def kernel(x, edge_index, W, b):
    src = edge_index[0]
    dst = edge_index[1]
    zeros = jnp.zeros((NROWS, D), jnp.float32)

    dout, din = _deg_kernel(src, dst)
    dout = dout.reshape(NC, NROWS, 1)
    din = din.reshape(NC, NROWS, 1)

    h = _mm_scale(x, W, dout)
    agg = _agg_kernel(h, src, dst, zeros)
    agg = agg.reshape(NC, NROWS, D)
    out = _final(agg, h, din, b.reshape(1, D))
    return out
